# Initial kernel scaffold; baseline (speedup 1.0000x reference)
import jax
import jax.numpy as jnp
from jax.experimental import pallas as pl


def kernel(x, tag):
    raise NotImplementedError("write your pallas kernel here")


# speedup vs baseline: 1.2767x; 1.2767x over previous
import jax
import jax.numpy as jnp
from jax.experimental import pallas as pl
from jax.experimental.pallas import tpu as pltpu

_ALPHA = 0.25
_BLOCK_ROWS = 256


def _focal_body(x_ref, t_ref, o_ref, sum_ref, cnt_ref):
    i = pl.program_id(0)

    @pl.when(i == 0)
    def _init():
        sum_ref[0] = 0.0
        cnt_ref[0] = 0.0

    d = 1.0 - x_ref[...]
    loss = d * d
    m = t_ref[...] == 1
    sum_ref[0] += jnp.sum(jnp.where(m, loss, 0.0))
    cnt_ref[0] += jnp.sum(m.astype(jnp.float32))

    @pl.when(i == pl.num_programs(0) - 1)
    def _fini():
        o_ref[0, 0] = (_ALPHA * sum_ref[0]) / cnt_ref[0]


def kernel(x, tag):
    rows, cols = x.shape
    grid = rows // _BLOCK_ROWS
    out = pl.pallas_call(
        _focal_body,
        grid=(grid,),
        in_specs=[
            pl.BlockSpec((_BLOCK_ROWS, cols), lambda i: (i, 0)),
            pl.BlockSpec((_BLOCK_ROWS, cols), lambda i: (i, 0)),
        ],
        out_specs=pl.BlockSpec(memory_space=pltpu.SMEM),
        out_shape=jax.ShapeDtypeStruct((1, 1), x.dtype),
        scratch_shapes=[
            pltpu.SMEM((1,), jnp.float32),
            pltpu.SMEM((1,), jnp.float32),
        ],
        compiler_params=pltpu.CompilerParams(
            dimension_semantics=("arbitrary",),
        ),
    )(x, tag)
    return out.reshape(())


# speedup vs baseline: 1.2899x; 1.0103x over previous
import jax
import jax.numpy as jnp
from jax.experimental import pallas as pl
from jax.experimental.pallas import tpu as pltpu

_ALPHA = 0.25
_BLOCK_ROWS = 512


def _focal_body(x_ref, t_ref, o_ref, sum_ref, cnt_ref):
    i = pl.program_id(0)

    @pl.when(i == 0)
    def _init():
        sum_ref[0] = 0.0
        cnt_ref[0] = 0.0

    d = 1.0 - x_ref[...]
    loss = d * d
    m = t_ref[...] == 1
    sum_ref[0] += jnp.sum(jnp.where(m, loss, 0.0))
    cnt_ref[0] += jnp.sum(m.astype(jnp.float32))

    @pl.when(i == pl.num_programs(0) - 1)
    def _fini():
        o_ref[0, 0] = (_ALPHA * sum_ref[0]) / cnt_ref[0]


def kernel(x, tag):
    rows, cols = x.shape
    grid = rows // _BLOCK_ROWS
    out = pl.pallas_call(
        _focal_body,
        grid=(grid,),
        in_specs=[
            pl.BlockSpec((_BLOCK_ROWS, cols), lambda i: (i, 0)),
            pl.BlockSpec((_BLOCK_ROWS, cols), lambda i: (i, 0)),
        ],
        out_specs=pl.BlockSpec(memory_space=pltpu.SMEM),
        out_shape=jax.ShapeDtypeStruct((1, 1), x.dtype),
        scratch_shapes=[
            pltpu.SMEM((1,), jnp.float32),
            pltpu.SMEM((1,), jnp.float32),
        ],
        compiler_params=pltpu.CompilerParams(
            dimension_semantics=("arbitrary",),
        ),
    )(x, tag)
    return out.reshape(())
